# baseline (speedup 1.0000x reference)
import functools

import jax
import jax.numpy as jnp
from jax import lax
from jax.experimental import pallas as pl
from jax.experimental.pallas import tpu as pltpu
from jax.experimental.pallas import tpu_sc as plsc

_Q = 0.7
_K = 0.5
_C2 = (1.0 - _K ** _Q) / _Q


def _sc_dense(logits, targets, indexes, weight, b_tc):
    B, C = logits.shape
    nsc = B - b_tc
    info = plsc.get_sparse_core_info()
    nw = info.num_cores * info.num_subcores
    rpw = nsc // nw
    nch = rpw // 16
    gpw = B // nw
    mesh = plsc.VectorSubcoreMesh(core_axis_name="c", subcore_axis_name="s")

    @functools.partial(
        pl.kernel,
        mesh=mesh,
        out_type=(
            jax.ShapeDtypeStruct((nsc,), jnp.float32),
            jax.ShapeDtypeStruct((nsc,), jnp.float32),
            jax.ShapeDtypeStruct((B,), jnp.float32),
        ),
        scratch_types=[
            pltpu.VMEM((rpw,), jnp.int32),
            pltpu.VMEM((gpw,), jnp.int32),
            pltpu.VMEM((gpw,), jnp.float32),
            pltpu.VMEM((2, 16, C), jnp.float32),
            pltpu.VMEM((rpw,), jnp.float32),
            pltpu.VMEM((rpw,), jnp.float32),
            pltpu.SemaphoreType.DMA,
            pltpu.SemaphoreType.DMA,
            pltpu.SemaphoreType.DMA,
        ],
        compiler_params=pltpu.CompilerParams(needs_layout_passes=False),
    )
    def k(logits_hbm, targets_hbm, indexes_hbm, weight_hbm,
          u_hbm, s_hbm, w_hbm,
          tgt_v, idx_v, w_v, buf, u_v, s_v, sem0, sem1, semw):
        wid = lax.axis_index("s") * info.num_cores + lax.axis_index("c")
        gbase = wid * gpw
        pltpu.sync_copy(indexes_hbm.at[pl.ds(gbase, gpw)], idx_v)
        gdesc = pltpu.async_copy(weight_hbm.at[idx_v], w_v, semw)

        base = b_tc + wid * rpw
        pltpu.sync_copy(targets_hbm.at[pl.ds(base, rpw)], tgt_v)

        sems = (sem0, sem1)

        def start(g):
            return pltpu.async_copy(
                logits_hbm.at[pl.ds(base + g * 16, 16), :], buf.at[g % 2],
                sems[g % 2])

        descs = {0: start(0)}
        rows = lax.broadcasted_iota(jnp.int32, (16,), 0)
        for g in range(nch):
            if g + 1 < nch:
                descs[g + 1] = start(g + 1)
            descs[g].wait()
            bufg = buf.at[g % 2]

            def p1(c, m, bufg=bufg):
                v = plsc.load_gather(bufg, [rows, jnp.full((16,), 0, jnp.int32) + c])
                return jnp.maximum(m, v)

            m = lax.fori_loop(0, C, p1, jnp.full((16,), -jnp.inf, jnp.float32),
                              unroll=8)
            tgt16 = tgt_v[pl.ds(g * 16, 16)]
            lt = plsc.load_gather(bufg, [rows, tgt16])

            def p2(c, s, bufg=bufg, m=m):
                v = plsc.load_gather(bufg, [rows, jnp.full((16,), 0, jnp.int32) + c])
                return s + jnp.exp(v - m)

            s = lax.fori_loop(0, C, p2, jnp.zeros((16,), jnp.float32),
                              unroll=8)
            u_v[pl.ds(g * 16, 16)] = lt - m
            s_v[pl.ds(g * 16, 16)] = s

        pltpu.sync_copy(u_v, u_hbm.at[pl.ds(wid * rpw, rpw)])
        pltpu.sync_copy(s_v, s_hbm.at[pl.ds(wid * rpw, rpw)])
        gdesc.wait()
        pltpu.sync_copy(w_v, w_hbm.at[pl.ds(gbase, gpw)])

    return k(logits, targets, indexes, weight.reshape(-1))


def _combine_body(inv_b, u_ref, s_ref, w_ref, o_ref):
    u = u_ref[:, :]
    s = s_ref[:, :]
    w = w_ref[:, :]
    g = (1.0 - jnp.exp(_Q * (u - jnp.log(s)))) / _Q - _C2
    o_ref[0, 0] = jnp.sum(g * w) * inv_b


def _combine(u, s, w, B):
    n = u.shape[0]
    u2 = u.reshape(n // 128, 128)
    s2 = s.reshape(n // 128, 128)
    w2 = w.reshape(n // 128, 128)
    return pl.pallas_call(
        functools.partial(_combine_body, 1.0 / B),
        out_specs=pl.BlockSpec(memory_space=pltpu.SMEM),
        out_shape=jax.ShapeDtypeStruct((1, 1), jnp.float32),
    )(u2, s2, w2)


@jax.jit
def kernel(logits, targets, indexes, weight):
    B = logits.shape[0]
    u, s, w = _sc_dense(logits, targets, indexes, weight, 0)
    out = _combine(u, s, w, B)
    return out[0, 0]


# speedup vs baseline: 3.0775x; 3.0775x over previous
import functools

import jax
import jax.numpy as jnp
from jax import lax
from jax.experimental import pallas as pl
from jax.experimental.pallas import tpu as pltpu
from jax.experimental.pallas import tpu_sc as plsc

_Q = 0.7
_K = 0.5
_C2 = (1.0 - _K ** _Q) / _Q


def _sc_gather(table, idx):
    B = idx.shape[0]
    info = plsc.get_sparse_core_info()
    nw = info.num_cores * info.num_subcores
    bpw = B // nw
    mesh = plsc.VectorSubcoreMesh(core_axis_name="c", subcore_axis_name="s")

    @functools.partial(
        pl.kernel,
        mesh=mesh,
        out_type=jax.ShapeDtypeStruct((B,), jnp.float32),
        scratch_types=[
            pltpu.VMEM((bpw,), jnp.int32),
            pltpu.VMEM((bpw,), jnp.float32),
            pltpu.SemaphoreType.DMA,
        ],
    )
    def k(table_hbm, idx_hbm, out_hbm, idx_v, rows_v, sem):
        wid = lax.axis_index("s") * info.num_cores + lax.axis_index("c")
        base = wid * bpw
        pltpu.sync_copy(idx_hbm.at[pl.ds(base, bpw)], idx_v)
        pltpu.async_copy(table_hbm.at[idx_v], rows_v, sem).wait()
        pltpu.sync_copy(rows_v, out_hbm.at[pl.ds(base, bpw)])

    return k(table, idx)


def _g_of(x, t2d):
    m = jnp.max(x, axis=1, keepdims=True)
    s = jnp.sum(jnp.exp(x - m), axis=1, keepdims=True)
    cols = lax.broadcasted_iota(jnp.int32, x.shape, 1)
    lt = jnp.sum(jnp.where(cols == t2d, x, 0.0), axis=1, keepdims=True)
    log_yg = lt - m - jnp.log(s)
    return (1.0 - jnp.exp(_Q * log_yg)) / _Q - _C2


def _tc_dense_body(x1, x2, x3, x4, t1, t2, t3, t4, g1, g2, g3, g4):
    g1[:, :] = _g_of(x1[:, :], t1[:, :])
    g2[:, :] = _g_of(x2[:, :], t2[:, :])
    g3[:, :] = _g_of(x3[:, :], t3[:, :])
    g4[:, :] = _g_of(x4[:, :], t4[:, :])


def _tc_dense(logits, targets2d):
    B, C = logits.shape
    R = 512
    nsteps = B // 4 // R
    quarter = B // 4
    x_spec = [pl.BlockSpec((R, C), functools.partial(
        lambda q, i: (i + q * nsteps, 0), q)) for q in range(4)]
    t_spec = [pl.BlockSpec((R, 1), functools.partial(
        lambda q, i: (i + q * nsteps, 0), q)) for q in range(4)]
    g_spec = [pl.BlockSpec((R, 1), lambda i: (i, 0)) for _ in range(4)]
    g_shape = [jax.ShapeDtypeStruct((quarter, 1), jnp.float32)] * 4
    return pl.pallas_call(
        _tc_dense_body,
        grid=(nsteps,),
        in_specs=x_spec + t_spec,
        out_specs=g_spec,
        out_shape=g_shape,
    )(logits, logits, logits, logits, targets2d, targets2d, targets2d,
      targets2d)


def _combine_body(inv_b, g1, g2, g3, g4, w_ref, o_ref):
    q = g1.shape[0]
    tot = (jnp.sum(g1[:, :] * w_ref[0 * q:1 * q, :])
           + jnp.sum(g2[:, :] * w_ref[1 * q:2 * q, :])
           + jnp.sum(g3[:, :] * w_ref[2 * q:3 * q, :])
           + jnp.sum(g4[:, :] * w_ref[3 * q:4 * q, :]))
    o_ref[0, 0] = tot * inv_b


def _combine(gs, w2d, B):
    return pl.pallas_call(
        functools.partial(_combine_body, 1.0 / B),
        out_specs=pl.BlockSpec(memory_space=pltpu.SMEM),
        out_shape=jax.ShapeDtypeStruct((1, 1), jnp.float32),
    )(*gs, w2d)


@jax.jit
def kernel(logits, targets, indexes, weight):
    B = logits.shape[0]
    w = _sc_gather(weight.reshape(-1), indexes)
    gs = _tc_dense(logits, targets.reshape(B, 1))
    out = _combine(gs, w.reshape(B, 1), B)
    return out[0, 0]
